# Initial kernel scaffold; baseline (speedup 1.0000x reference)
import jax
import jax.numpy as jnp
from jax.experimental import pallas as pl


def kernel(cls_score_0, bbox_pred_0, cls_score_1, bbox_pred_1, cls_score_2, bbox_pred_2, origin_shapes):
    raise NotImplementedError("write your pallas kernel here")


# speedup vs baseline: 1.4274x; 1.4274x over previous
import jax
import jax.numpy as jnp
from jax import lax
from jax.experimental import pallas as pl
from jax.experimental.pallas import tpu as pltpu

B = 8
C = 80
REG = 7
NMS_PRE = 1000
CAP = 200
TOPK = 100
IOU_T = 0.5
BOX_SCORE = 0.3
IMG = 512.0
LEVELS = ((4096, 64, 8.0), (1024, 32, 16.0), (256, 16, 32.0))
NPOOL = NMS_PRE + NMS_PRE + 256
CC = 16
NFLAT = C * CAP
NEG = -1e30

_HI_KEY = 0x3F800000
_CHUNK = 250


def _isub(n):
    return lax.broadcasted_iota(jnp.int32, (n, 1), 0)


def _ilane(n):
    return lax.broadcasted_iota(jnp.int32, (1, n), 1)


def _cumsum_lanes(v):
    n = v.shape[-1]
    d = 1
    while d < n:
        pad = jnp.zeros(v[..., :d].shape, v.dtype)
        v = v + jnp.concatenate([pad, v[..., :n - d]], axis=-1)
        d *= 2
    return v


def _hot(cond):
    return jnp.where(cond, jnp.float32(1.0), jnp.float32(0.0))


def _mm(a, b):
    return lax.dot_general(a, b, (((1,), (0,)), ((), ())),
                           precision=lax.Precision.HIGHEST,
                           preferred_element_type=jnp.float32)


def _mm_c0(a, b):
    return lax.dot_general(a, b, (((0,), (0,)), ((), ())),
                           precision=lax.Precision.HIGHEST,
                           preferred_element_type=jnp.float32)


def _mm_bt(a, b):
    return lax.dot_general(a, b, (((1,), (1,)), ((), ())),
                           precision=lax.Precision.HIGHEST,
                           preferred_element_type=jnp.float32)


def _topk_select(key, k, lo0, hi0, iters):
    r = key.shape[0]
    lo = jnp.full((r, 1), lo0, jnp.int32)
    hi = jnp.full((r, 1), hi0, jnp.int32)

    def step(_, carry):
        lo, hi = carry
        mid = (lo + hi + 1) // 2
        cnt = jnp.sum((key >= mid).astype(jnp.int32), axis=1, keepdims=True)
        ok = cnt >= k
        return jnp.where(ok, mid, lo), jnp.where(ok, hi, mid - 1)

    lo, hi = lax.fori_loop(0, iters, step, (lo, hi))
    t = lo
    gt = key > t
    cnt_gt = jnp.sum(gt.astype(jnp.int32), axis=1, keepdims=True)
    eq = (key == t).astype(jnp.int32)
    eqx = _cumsum_lanes(eq) - eq
    sel = gt | ((eq > 0) & (eqx < (k - cnt_gt)))
    dest = _cumsum_lanes(sel.astype(jnp.int32)) - 1
    return sel, dest


def _rank_desc(kc_col, kc_row):
    n = kc_col.shape[0]
    isub, ilane = _isub(n), _ilane(n)
    cmp = _hot((kc_col > kc_row) | ((kc_col == kc_row) & (isub < ilane)))
    rank_row = jnp.sum(cmp, axis=0, keepdims=True).astype(jnp.int32)
    cmp_t = _hot((kc_row > kc_col) | ((kc_row == kc_col) & (ilane < isub)))
    rank_col = jnp.sum(cmp_t, axis=1, keepdims=True).astype(jnp.int32)
    return rank_row, rank_col


def _decode_boxes_t(bbt, hw, w, stride):
    rows = []
    for g in range(4):
        sub = bbt[8 * g:8 * g + 8, :]
        m = jnp.max(sub, axis=0, keepdims=True)
        e = jnp.exp(sub - m)
        s = e
        while s.shape[0] > 1:
            h = s.shape[0] // 2
            s = s[:h, :] + s[h:, :]
        sm = (e / s).astype(jnp.bfloat16).astype(jnp.float32)
        acc = sm[1:2, :]
        for i in range(2, REG + 1):
            acc = acc + sm[i:i + 1, :] * jnp.float32(i)
        rows.append(acc)
    dist = jnp.concatenate(rows, axis=0) * jnp.float32(stride)
    n = _ilane(hw)
    py = ((n // w).astype(jnp.float32) + 0.5) * jnp.float32(stride)
    px = ((n % w).astype(jnp.float32) + 0.5) * jnp.float32(stride)
    y1 = jnp.minimum(jnp.maximum(py - dist[0:1, :], 0.0), IMG)
    x1 = jnp.minimum(jnp.maximum(px - dist[1:2, :], 0.0), IMG)
    y2 = jnp.minimum(jnp.maximum(py + dist[2:3, :], 0.0), IMG)
    x2 = jnp.minimum(jnp.maximum(px + dist[3:4, :], 0.0), IMG)
    return jnp.concatenate([y1, x1, y2, x2], axis=0)


def _pool_body(c0, b0, c1, b1, c2, b2, out_s, out_b, comp_scr):
    row = 0
    for li, (hw, w, stride) in enumerate(LEVELS):
        cls = (c0, c1, c2)[li][0]
        bbt = jnp.transpose((b0, b1, b2)[li][0])
        scores = jax.nn.sigmoid(cls)
        boxes_t = _decode_boxes_t(bbt, hw, w, stride)
        boxes = jnp.transpose(boxes_t)
        if hw > NMS_PRE:
            maxs_col = jnp.max(scores, axis=1, keepdims=True)
            maxs_row = jnp.transpose(maxs_col)
            key = lax.bitcast_convert_type(maxs_row, jnp.int32)
            sel, dest = _topk_select(key, NMS_PRE, 0, _HI_KEY, 31)
            dat = jnp.concatenate([scores, boxes, maxs_col], axis=-1)

            def gather_chunk(t, _):
                j0 = t * _CHUNK
                p1c = _hot(((_isub(_CHUNK) + j0) == dest) & sel)
                comp_scr[pl.ds(j0, _CHUNK), :] = _mm(p1c, dat)
                return 0

            lax.fori_loop(0, NMS_PRE // _CHUNK, gather_chunk, 0)
            compact = comp_scr[...]
            kc_col = compact[:, C + 4:C + 5]
            kc_row = jnp.transpose(kc_col)
            rank_row, _ = _rank_desc(kc_col, kc_row)
            p2 = _hot(_isub(NMS_PRE) == rank_row)
            ordered = _mm(p2, compact)
            out_s[0, row:row + NMS_PRE, :] = ordered[:, :C]
            out_b[0, row:row + NMS_PRE, :] = ordered[:, C:C + 4]
            row += NMS_PRE
        else:
            out_s[0, row:row + hw, :] = scores
            out_b[0, row:row + hw, :] = boxes
            row += hw


def _nms_body(st, pbt, out_bt, sel_scr, dest_scr, bx_scr, bt_scr, iou_scr):
    rows = st[0]
    key = lax.bitcast_convert_type(rows, jnp.int32)
    sel, dest = _topk_select(key, CAP, 0, _HI_KEY, 31)
    sel_scr[...] = sel.astype(jnp.int32)
    dest_scr[...] = dest
    pbtv = pbt[0]

    def per_class(c, _):
        dest_row = dest_scr[pl.ds(c, 1), :]
        sel_row = sel_scr[pl.ds(c, 1), :]
        rows_c = st[0, pl.ds(c, 1), :]
        gh = _hot((_isub(CAP) == dest_row) & (sel_row > 0))
        datt = jnp.concatenate([pbtv, rows_c], axis=0)
        compact = _mm_bt(gh, datt)
        kc_col = compact[:, 4:5]
        kc_row = jnp.transpose(kc_col)
        rank_row, rank_col = _rank_desc(kc_col, kc_row)
        p2 = _hot(_isub(CAP) == rank_row)
        ordered = _mm(p2, compact)
        p2r = _hot(rank_col == _ilane(CAP))
        ordered_t = _mm_c0(compact, p2r)
        bx_scr[pl.ds(c, 1), :, :] = ordered[jnp.newaxis, :, :4]
        bt_scr[pl.ds(c, 1), :, :] = ordered_t[jnp.newaxis]
        return 0

    lax.fori_loop(0, CC, per_class, 0)
    bxv = bx_scr[...]
    btv = bt_scr[...]
    y1c, x1c = bxv[:, :, 0:1], bxv[:, :, 1:2]
    y2c, x2c = bxv[:, :, 2:3], bxv[:, :, 3:4]
    y1r, x1r = btv[:, 0:1, :], btv[:, 1:2, :]
    y2r, x2r = btv[:, 2:3, :], btv[:, 3:4, :]
    area_c = jnp.maximum(y2c - y1c, 0.0) * jnp.maximum(x2c - x1c, 0.0)
    area_r = jnp.maximum(y2r - y1r, 0.0) * jnp.maximum(x2r - x1r, 0.0)
    inter = (jnp.maximum(jnp.minimum(y2c, y2r) - jnp.maximum(y1c, y1r), 0.0)
             * jnp.maximum(jnp.minimum(x2c, x2r) - jnp.maximum(x1c, x1r), 0.0))
    iou_scr[...] = inter / jnp.maximum(area_c + area_r - inter, 1e-8)

    lane = lax.broadcasted_iota(jnp.int32, (CC, CAP), 1)

    def step(i, supp):
        hit = jnp.sum(jnp.where((lane == i) & (supp > 0), 1, 0),
                      axis=1, keepdims=True)
        kept = hit == 0
        iou_row = iou_scr[:, pl.ds(i, 1), :].reshape(CC, CAP)
        return supp | (kept & (iou_row > IOU_T) & (lane > i)).astype(jnp.int32)

    supp = lax.fori_loop(0, CAP, step, jnp.zeros((CC, CAP), jnp.int32))
    keep = supp == 0
    keep = keep & (_cumsum_lanes(keep.astype(jnp.int32)) <= TOPK)
    sel_scores = jnp.where(keep, btv[:, 4, :], jnp.float32(NEG))
    for c in range(CC):
        out_bt[0, 0:4, c * CAP:(c + 1) * CAP] = btv[c, 0:4, :]
        out_bt[0, 4:5, c * CAP:(c + 1) * CAP] = sel_scores[c:c + 1, :]


def _final_body(fdt, out):
    dt = fdt[0]
    row = dt[4:5, :]
    key = jnp.where(row == jnp.float32(NEG),
                    jnp.int32(-1),
                    lax.bitcast_convert_type(row, jnp.int32))
    sel, dest = _topk_select(key, TOPK, -1, _HI_KEY, 31)
    cls_row = (_ilane(NFLAT) // CAP).astype(jnp.float32)
    datt = jnp.concatenate([dt, cls_row], axis=0)
    gh = _hot((_isub(TOPK) == dest) & sel)
    compact = _mm_bt(gh, datt)
    kc_col = compact[:, 4:5]
    kc_row = jnp.transpose(kc_col)
    rank_row, _ = _rank_desc(kc_col, kc_row)
    p2 = _hot(_isub(TOPK) == rank_row)
    ordered = _mm(p2, compact)
    mask = _hot(ordered[:, 4:5] > BOX_SCORE)
    out[0] = ordered * mask


def kernel(cls_score_0, bbox_pred_0, cls_score_1, bbox_pred_1,
           cls_score_2, bbox_pred_2, origin_shapes):
    del origin_shapes
    c0 = cls_score_0.reshape(B, 4096, C)
    b0 = bbox_pred_0.reshape(B, 4096, 4 * (REG + 1))
    c1 = cls_score_1.reshape(B, 1024, C)
    b1 = bbox_pred_1.reshape(B, 1024, 4 * (REG + 1))
    c2 = cls_score_2.reshape(B, 256, C)
    b2 = bbox_pred_2.reshape(B, 256, 4 * (REG + 1))

    pool_s, pool_b = pl.pallas_call(
        _pool_body,
        grid=(B,),
        in_specs=[
            pl.BlockSpec((1, 4096, C), lambda i: (i, 0, 0)),
            pl.BlockSpec((1, 4096, 32), lambda i: (i, 0, 0)),
            pl.BlockSpec((1, 1024, C), lambda i: (i, 0, 0)),
            pl.BlockSpec((1, 1024, 32), lambda i: (i, 0, 0)),
            pl.BlockSpec((1, 256, C), lambda i: (i, 0, 0)),
            pl.BlockSpec((1, 256, 32), lambda i: (i, 0, 0)),
        ],
        out_specs=[
            pl.BlockSpec((1, NPOOL, C), lambda i: (i, 0, 0)),
            pl.BlockSpec((1, NPOOL, 4), lambda i: (i, 0, 0)),
        ],
        out_shape=[
            jax.ShapeDtypeStruct((B, NPOOL, C), jnp.float32),
            jax.ShapeDtypeStruct((B, NPOOL, 4), jnp.float32),
        ],
        scratch_shapes=[pltpu.VMEM((NMS_PRE, C + 5), jnp.float32)],
    )(c0, b0, c1, b1, c2, b2)

    pool_st = pool_s.transpose(0, 2, 1)
    pool_bt = pool_b.transpose(0, 2, 1)

    flat_bt = pl.pallas_call(
        _nms_body,
        grid=(B, C // CC),
        in_specs=[
            pl.BlockSpec((1, CC, NPOOL), lambda i, k: (i, k, 0)),
            pl.BlockSpec((1, 4, NPOOL), lambda i, k: (i, 0, 0)),
        ],
        out_specs=pl.BlockSpec((1, 5, CC * CAP), lambda i, k: (i, 0, k)),
        out_shape=jax.ShapeDtypeStruct((B, 5, NFLAT), jnp.float32),
        scratch_shapes=[
            pltpu.VMEM((CC, NPOOL), jnp.int32),
            pltpu.VMEM((CC, NPOOL), jnp.int32),
            pltpu.VMEM((CC, CAP, 4), jnp.float32),
            pltpu.VMEM((CC, 5, CAP), jnp.float32),
            pltpu.VMEM((CC, CAP, CAP), jnp.float32),
        ],
    )(pool_st, pool_bt)

    res = pl.pallas_call(
        _final_body,
        grid=(B,),
        in_specs=[pl.BlockSpec((1, 5, NFLAT), lambda i: (i, 0, 0))],
        out_specs=pl.BlockSpec((1, TOPK, 6), lambda i: (i, 0, 0)),
        out_shape=jax.ShapeDtypeStruct((B, TOPK, 6), jnp.float32),
    )(flat_bt)
    return res


# speedup vs baseline: 1.7541x; 1.2288x over previous
import jax
import jax.numpy as jnp
from jax import lax
from jax.experimental import pallas as pl
from jax.experimental.pallas import tpu as pltpu

B = 8
C = 80
REG = 7
NMS_PRE = 1000
CAP = 200
TOPK = 100
IOU_T = 0.5
BOX_SCORE = 0.3
IMG = 512.0
LEVELS = ((4096, 64, 8.0), (1024, 32, 16.0), (256, 16, 32.0))
NPOOL = NMS_PRE + NMS_PRE + 256
CC = 80
NFLAT = C * CAP
NEG = -1e30

_HI_KEY = 0x3F800000
_CHUNK = 250


def _isub(n):
    return lax.broadcasted_iota(jnp.int32, (n, 1), 0)


def _ilane(n):
    return lax.broadcasted_iota(jnp.int32, (1, n), 1)


def _cumsum_lanes(v):
    n = v.shape[-1]
    d = 1
    while d < n:
        pad = jnp.zeros(v[..., :d].shape, v.dtype)
        v = v + jnp.concatenate([pad, v[..., :n - d]], axis=-1)
        d *= 2
    return v


def _hot(cond):
    return jnp.where(cond, jnp.float32(1.0), jnp.float32(0.0))


def _mm(a, b):
    return lax.dot_general(a, b, (((1,), (0,)), ((), ())),
                           precision=lax.Precision.HIGHEST,
                           preferred_element_type=jnp.float32)


def _mm_c0(a, b):
    return lax.dot_general(a, b, (((0,), (0,)), ((), ())),
                           precision=lax.Precision.HIGHEST,
                           preferred_element_type=jnp.float32)


def _mm_bt(a, b):
    return lax.dot_general(a, b, (((1,), (1,)), ((), ())),
                           precision=lax.Precision.HIGHEST,
                           preferred_element_type=jnp.float32)


def _topk_select(key, k, lo0, hi0, iters):
    r = key.shape[0]
    lo = jnp.full((r, 1), lo0, jnp.int32)
    hi = jnp.full((r, 1), hi0, jnp.int32)

    def step(_, carry):
        lo, hi = carry
        mid = (lo + hi + 1) // 2
        cnt = jnp.sum((key >= mid).astype(jnp.int32), axis=1, keepdims=True)
        ok = cnt >= k
        return jnp.where(ok, mid, lo), jnp.where(ok, hi, mid - 1)

    lo, hi = lax.fori_loop(0, iters, step, (lo, hi))
    t = lo
    gt = key > t
    cnt_gt = jnp.sum(gt.astype(jnp.int32), axis=1, keepdims=True)
    eq = (key == t).astype(jnp.int32)
    eqx = _cumsum_lanes(eq) - eq
    sel = gt | ((eq > 0) & (eqx < (k - cnt_gt)))
    dest = _cumsum_lanes(sel.astype(jnp.int32)) - 1
    return sel, dest


def _rank_desc(kc_col, kc_row):
    n = kc_col.shape[0]
    isub, ilane = _isub(n), _ilane(n)
    cmp = _hot((kc_col > kc_row) | ((kc_col == kc_row) & (isub < ilane)))
    rank_row = jnp.sum(cmp, axis=0, keepdims=True).astype(jnp.int32)
    cmp_t = _hot((kc_row > kc_col) | ((kc_row == kc_col) & (ilane < isub)))
    rank_col = jnp.sum(cmp_t, axis=1, keepdims=True).astype(jnp.int32)
    return rank_row, rank_col


def _decode_boxes_t(bbt, hw, w, stride):
    rows = []
    for g in range(4):
        sub = bbt[8 * g:8 * g + 8, :]
        m = jnp.max(sub, axis=0, keepdims=True)
        e = jnp.exp(sub - m)
        s = e
        while s.shape[0] > 1:
            h = s.shape[0] // 2
            s = s[:h, :] + s[h:, :]
        sm = (e / s).astype(jnp.bfloat16).astype(jnp.float32)
        acc = sm[1:2, :]
        for i in range(2, REG + 1):
            acc = acc + sm[i:i + 1, :] * jnp.float32(i)
        rows.append(acc)
    dist = jnp.concatenate(rows, axis=0) * jnp.float32(stride)
    n = _ilane(hw)
    py = ((n // w).astype(jnp.float32) + 0.5) * jnp.float32(stride)
    px = ((n % w).astype(jnp.float32) + 0.5) * jnp.float32(stride)
    y1 = jnp.minimum(jnp.maximum(py - dist[0:1, :], 0.0), IMG)
    x1 = jnp.minimum(jnp.maximum(px - dist[1:2, :], 0.0), IMG)
    y2 = jnp.minimum(jnp.maximum(py + dist[2:3, :], 0.0), IMG)
    x2 = jnp.minimum(jnp.maximum(px + dist[3:4, :], 0.0), IMG)
    return jnp.concatenate([y1, x1, y2, x2], axis=0)


def _pool_body(c0, b0, c1, b1, c2, b2, out_s, out_b, comp_scr):
    row = 0
    for li, (hw, w, stride) in enumerate(LEVELS):
        cls = (c0, c1, c2)[li][0]
        bbt = jnp.transpose((b0, b1, b2)[li][0])
        scores = jax.nn.sigmoid(cls)
        boxes_t = _decode_boxes_t(bbt, hw, w, stride)
        boxes = jnp.transpose(boxes_t)
        if hw > NMS_PRE:
            maxs_col = jnp.max(scores, axis=1, keepdims=True)
            maxs_row = jnp.transpose(maxs_col)
            key = lax.bitcast_convert_type(maxs_row, jnp.int32)
            sel, dest = _topk_select(key, NMS_PRE, 0, _HI_KEY, 31)
            dat = jnp.concatenate([scores, boxes, maxs_col], axis=-1)

            def gather_chunk(t, _):
                j0 = t * _CHUNK
                p1c = _hot(((_isub(_CHUNK) + j0) == dest) & sel)
                comp_scr[pl.ds(j0, _CHUNK), :] = _mm(p1c, dat)
                return 0

            lax.fori_loop(0, NMS_PRE // _CHUNK, gather_chunk, 0)
            compact = comp_scr[...]
            kc_col = compact[:, C + 4:C + 5]
            kc_row = jnp.transpose(kc_col)
            rank_row, _ = _rank_desc(kc_col, kc_row)
            p2 = _hot(_isub(NMS_PRE) == rank_row)
            ordered = _mm(p2, compact)
            out_s[0, row:row + NMS_PRE, :] = ordered[:, :C]
            out_b[0, row:row + NMS_PRE, :] = ordered[:, C:C + 4]
            row += NMS_PRE
        else:
            out_s[0, row:row + hw, :] = scores
            out_b[0, row:row + hw, :] = boxes
            row += hw


def _nms_body(st, pbt, out_bt, sel_scr, dest_scr, bx_scr, bt_scr, iou_scr):
    rows = st[0]
    key = lax.bitcast_convert_type(rows, jnp.int32)
    sel, dest = _topk_select(key, CAP, 0, _HI_KEY, 31)
    sel_scr[...] = sel.astype(jnp.int32)
    dest_scr[...] = dest
    pbtv = pbt[0]

    def per_class(c, _):
        dest_row = dest_scr[pl.ds(c, 1), :]
        sel_row = sel_scr[pl.ds(c, 1), :]
        rows_c = st[0, pl.ds(c, 1), :]
        gh = _hot((_isub(CAP) == dest_row) & (sel_row > 0))
        datt = jnp.concatenate([pbtv, rows_c], axis=0)
        compact = _mm_bt(gh, datt)
        kc_col = compact[:, 4:5]
        kc_row = jnp.transpose(kc_col)
        rank_row, rank_col = _rank_desc(kc_col, kc_row)
        p2 = _hot(_isub(CAP) == rank_row)
        ordered = _mm(p2, compact)
        p2r = _hot(rank_col == _ilane(CAP))
        ordered_t = _mm_c0(compact, p2r)
        bx_scr[pl.ds(c, 1), :, :] = ordered[jnp.newaxis, :, :4]
        bt_scr[pl.ds(c, 1), :, :] = ordered_t[jnp.newaxis]
        return 0

    lax.fori_loop(0, CC, per_class, 0)
    bxv = bx_scr[...]
    btv = bt_scr[...]
    y1c, x1c = bxv[:, :, 0:1], bxv[:, :, 1:2]
    y2c, x2c = bxv[:, :, 2:3], bxv[:, :, 3:4]
    y1r, x1r = btv[:, 0:1, :], btv[:, 1:2, :]
    y2r, x2r = btv[:, 2:3, :], btv[:, 3:4, :]
    area_c = jnp.maximum(y2c - y1c, 0.0) * jnp.maximum(x2c - x1c, 0.0)
    area_r = jnp.maximum(y2r - y1r, 0.0) * jnp.maximum(x2r - x1r, 0.0)
    inter = (jnp.maximum(jnp.minimum(y2c, y2r) - jnp.maximum(y1c, y1r), 0.0)
             * jnp.maximum(jnp.minimum(x2c, x2r) - jnp.maximum(x1c, x1r), 0.0))
    iou_scr[...] = inter / jnp.maximum(area_c + area_r - inter, 1e-8)

    lane = lax.broadcasted_iota(jnp.int32, (CC, CAP), 1)

    def step(i, supp):
        hit = jnp.sum(jnp.where((lane == i) & (supp > 0), 1, 0),
                      axis=1, keepdims=True)
        kept = hit == 0
        iou_row = iou_scr[:, pl.ds(i, 1), :].reshape(CC, CAP)
        return supp | (kept & (iou_row > IOU_T) & (lane > i)).astype(jnp.int32)

    supp = lax.fori_loop(0, CAP, step, jnp.zeros((CC, CAP), jnp.int32))
    keep = supp == 0
    keep = keep & (_cumsum_lanes(keep.astype(jnp.int32)) <= TOPK)
    sel_scores = jnp.where(keep, btv[:, 4, :], jnp.float32(NEG))
    for c in range(CC):
        out_bt[0, 0:4, c * CAP:(c + 1) * CAP] = btv[c, 0:4, :]
        out_bt[0, 4:5, c * CAP:(c + 1) * CAP] = sel_scores[c:c + 1, :]


def _final_body(fdt, out):
    dt = fdt[0]
    row = dt[4:5, :]
    key = jnp.where(row == jnp.float32(NEG),
                    jnp.int32(-1),
                    lax.bitcast_convert_type(row, jnp.int32))
    sel, dest = _topk_select(key, TOPK, -1, _HI_KEY, 31)
    cls_row = (_ilane(NFLAT) // CAP).astype(jnp.float32)
    datt = jnp.concatenate([dt, cls_row], axis=0)
    gh = _hot((_isub(TOPK) == dest) & sel)
    compact = _mm_bt(gh, datt)
    kc_col = compact[:, 4:5]
    kc_row = jnp.transpose(kc_col)
    rank_row, _ = _rank_desc(kc_col, kc_row)
    p2 = _hot(_isub(TOPK) == rank_row)
    ordered = _mm(p2, compact)
    mask = _hot(ordered[:, 4:5] > BOX_SCORE)
    out[0] = ordered * mask


def kernel(cls_score_0, bbox_pred_0, cls_score_1, bbox_pred_1,
           cls_score_2, bbox_pred_2, origin_shapes):
    del origin_shapes
    c0 = cls_score_0.reshape(B, 4096, C)
    b0 = bbox_pred_0.reshape(B, 4096, 4 * (REG + 1))
    c1 = cls_score_1.reshape(B, 1024, C)
    b1 = bbox_pred_1.reshape(B, 1024, 4 * (REG + 1))
    c2 = cls_score_2.reshape(B, 256, C)
    b2 = bbox_pred_2.reshape(B, 256, 4 * (REG + 1))

    pool_s, pool_b = pl.pallas_call(
        _pool_body,
        grid=(B,),
        in_specs=[
            pl.BlockSpec((1, 4096, C), lambda i: (i, 0, 0)),
            pl.BlockSpec((1, 4096, 32), lambda i: (i, 0, 0)),
            pl.BlockSpec((1, 1024, C), lambda i: (i, 0, 0)),
            pl.BlockSpec((1, 1024, 32), lambda i: (i, 0, 0)),
            pl.BlockSpec((1, 256, C), lambda i: (i, 0, 0)),
            pl.BlockSpec((1, 256, 32), lambda i: (i, 0, 0)),
        ],
        out_specs=[
            pl.BlockSpec((1, NPOOL, C), lambda i: (i, 0, 0)),
            pl.BlockSpec((1, NPOOL, 4), lambda i: (i, 0, 0)),
        ],
        out_shape=[
            jax.ShapeDtypeStruct((B, NPOOL, C), jnp.float32),
            jax.ShapeDtypeStruct((B, NPOOL, 4), jnp.float32),
        ],
        scratch_shapes=[pltpu.VMEM((NMS_PRE, C + 5), jnp.float32)],
    )(c0, b0, c1, b1, c2, b2)

    pool_st = pool_s.transpose(0, 2, 1)
    pool_bt = pool_b.transpose(0, 2, 1)

    flat_bt = pl.pallas_call(
        _nms_body,
        grid=(B, C // CC),
        in_specs=[
            pl.BlockSpec((1, CC, NPOOL), lambda i, k: (i, k, 0)),
            pl.BlockSpec((1, 4, NPOOL), lambda i, k: (i, 0, 0)),
        ],
        out_specs=pl.BlockSpec((1, 5, CC * CAP), lambda i, k: (i, 0, k)),
        out_shape=jax.ShapeDtypeStruct((B, 5, NFLAT), jnp.float32),
        scratch_shapes=[
            pltpu.VMEM((CC, NPOOL), jnp.int32),
            pltpu.VMEM((CC, NPOOL), jnp.int32),
            pltpu.VMEM((CC, CAP, 4), jnp.float32),
            pltpu.VMEM((CC, 5, CAP), jnp.float32),
            pltpu.VMEM((CC, CAP, CAP), jnp.float32),
        ],
    )(pool_st, pool_bt)

    res = pl.pallas_call(
        _final_body,
        grid=(B,),
        in_specs=[pl.BlockSpec((1, 5, NFLAT), lambda i: (i, 0, 0))],
        out_specs=pl.BlockSpec((1, TOPK, 6), lambda i: (i, 0, 0)),
        out_shape=jax.ShapeDtypeStruct((B, TOPK, 6), jnp.float32),
    )(flat_bt)
    return res


# speedup vs baseline: 1.7970x; 1.0245x over previous
import jax
import jax.numpy as jnp
from jax import lax
from jax.experimental import pallas as pl
from jax.experimental.pallas import tpu as pltpu

B = 8
C = 80
REG = 7
NMS_PRE = 1000
CAP = 200
TOPK = 100
IOU_T = 0.5
BOX_SCORE = 0.3
IMG = 512.0
LEVELS = ((4096, 64, 8.0), (1024, 32, 16.0), (256, 16, 32.0))
NPOOL = NMS_PRE + NMS_PRE + 256
CC = 80
NFLAT = C * CAP
NEG = -1e30

_HI_KEY = 0x3F800000
_CHUNK = 250


def _isub(n):
    return lax.broadcasted_iota(jnp.int32, (n, 1), 0)


def _ilane(n):
    return lax.broadcasted_iota(jnp.int32, (1, n), 1)


def _cumsum_lanes(v):
    n = v.shape[-1]
    d = 1
    while d < n:
        pad = jnp.zeros(v[..., :d].shape, v.dtype)
        v = v + jnp.concatenate([pad, v[..., :n - d]], axis=-1)
        d *= 2
    return v


def _hot(cond):
    return jnp.where(cond, jnp.float32(1.0), jnp.float32(0.0))


def _mm(a, b):
    return lax.dot_general(a, b, (((1,), (0,)), ((), ())),
                           precision=lax.Precision.HIGHEST,
                           preferred_element_type=jnp.float32)


def _mm_c0(a, b):
    return lax.dot_general(a, b, (((0,), (0,)), ((), ())),
                           precision=lax.Precision.HIGHEST,
                           preferred_element_type=jnp.float32)


def _mm_bt(a, b):
    return lax.dot_general(a, b, (((1,), (1,)), ((), ())),
                           precision=lax.Precision.HIGHEST,
                           preferred_element_type=jnp.float32)


def _topk_select(key, k, lo0, hi0, iters):
    r = key.shape[0]
    lo = jnp.full((r, 1), lo0, jnp.int32)
    hi = jnp.full((r, 1), hi0, jnp.int32)

    def step(_, carry):
        lo, hi = carry
        mid = (lo + hi + 1) // 2
        cnt = jnp.sum((key >= mid).astype(jnp.int32), axis=1, keepdims=True)
        ok = cnt >= k
        return jnp.where(ok, mid, lo), jnp.where(ok, hi, mid - 1)

    lo, hi = lax.fori_loop(0, iters, step, (lo, hi))
    t = lo
    gt = key > t
    cnt_gt = jnp.sum(gt.astype(jnp.int32), axis=1, keepdims=True)
    eq = (key == t).astype(jnp.int32)
    eqx = _cumsum_lanes(eq) - eq
    sel = gt | ((eq > 0) & (eqx < (k - cnt_gt)))
    dest = _cumsum_lanes(sel.astype(jnp.int32)) - 1
    return sel, dest


def _rank_desc(kc_col, kc_row):
    n = kc_col.shape[0]
    isub, ilane = _isub(n), _ilane(n)
    cmp = _hot((kc_col > kc_row) | ((kc_col == kc_row) & (isub < ilane)))
    rank_row = jnp.sum(cmp, axis=0, keepdims=True).astype(jnp.int32)
    cmp_t = _hot((kc_row > kc_col) | ((kc_row == kc_col) & (ilane < isub)))
    rank_col = jnp.sum(cmp_t, axis=1, keepdims=True).astype(jnp.int32)
    return rank_row, rank_col


def _decode_boxes_t(bbt, hw, w, stride):
    rows = []
    for g in range(4):
        sub = bbt[8 * g:8 * g + 8, :]
        m = jnp.max(sub, axis=0, keepdims=True)
        e = jnp.exp(sub - m)
        s = e
        while s.shape[0] > 1:
            h = s.shape[0] // 2
            s = s[:h, :] + s[h:, :]
        sm = (e / s).astype(jnp.bfloat16).astype(jnp.float32)
        acc = sm[1:2, :]
        for i in range(2, REG + 1):
            acc = acc + sm[i:i + 1, :] * jnp.float32(i)
        rows.append(acc)
    dist = jnp.concatenate(rows, axis=0) * jnp.float32(stride)
    n = _ilane(hw)
    py = ((n // w).astype(jnp.float32) + 0.5) * jnp.float32(stride)
    px = ((n % w).astype(jnp.float32) + 0.5) * jnp.float32(stride)
    y1 = jnp.minimum(jnp.maximum(py - dist[0:1, :], 0.0), IMG)
    x1 = jnp.minimum(jnp.maximum(px - dist[1:2, :], 0.0), IMG)
    y2 = jnp.minimum(jnp.maximum(py + dist[2:3, :], 0.0), IMG)
    x2 = jnp.minimum(jnp.maximum(px + dist[3:4, :], 0.0), IMG)
    return jnp.concatenate([y1, x1, y2, x2], axis=0)


def _pool_body(c0, b0, c1, b1, c2, b2, out_s, out_b, comp_scr):
    row = 0
    for li, (hw, w, stride) in enumerate(LEVELS):
        cls = (c0, c1, c2)[li][0]
        bbt = jnp.transpose((b0, b1, b2)[li][0])
        scores = jax.nn.sigmoid(cls)
        boxes_t = _decode_boxes_t(bbt, hw, w, stride)
        boxes = jnp.transpose(boxes_t)
        if hw > NMS_PRE:
            maxs_col = jnp.max(scores, axis=1, keepdims=True)
            maxs_row = jnp.transpose(maxs_col)
            key = lax.bitcast_convert_type(maxs_row, jnp.int32)
            sel, dest = _topk_select(key, NMS_PRE, 0, _HI_KEY, 31)
            dat = jnp.concatenate([scores, boxes, maxs_col], axis=-1)

            def gather_chunk(t, _):
                j0 = t * _CHUNK
                p1c = _hot(((_isub(_CHUNK) + j0) == dest) & sel)
                comp_scr[pl.ds(j0, _CHUNK), :] = _mm(p1c, dat)
                return 0

            lax.fori_loop(0, NMS_PRE // _CHUNK, gather_chunk, 0)
            compact = comp_scr[...]
            kc_col = compact[:, C + 4:C + 5]
            kc_row = jnp.transpose(kc_col)
            rank_row, _ = _rank_desc(kc_col, kc_row)
            p2 = _hot(_isub(NMS_PRE) == rank_row)
            ordered = _mm(p2, compact)
            out_s[0, row:row + NMS_PRE, :] = ordered[:, :C]
            out_b[0, row:row + NMS_PRE, :] = ordered[:, C:C + 4]
            row += NMS_PRE
        else:
            out_s[0, row:row + hw, :] = scores
            out_b[0, row:row + hw, :] = boxes
            row += hw


def _nms_body(st, pbt, out_bt, sel_scr, dest_scr, bx_scr, bt_scr, iou_scr):
    rows = st[0]
    key = lax.bitcast_convert_type(rows, jnp.int32)
    sel, dest = _topk_select(key, CAP, 0, _HI_KEY, 31)
    sel_scr[...] = sel.astype(jnp.int32)
    dest_scr[...] = dest
    pbtv = pbt[0]

    def per_class(c, _):
        dest_row = dest_scr[pl.ds(c, 1), :]
        sel_row = sel_scr[pl.ds(c, 1), :]
        rows_c = st[0, pl.ds(c, 1), :]
        gh = _hot((_isub(CAP) == dest_row) & (sel_row > 0))
        datt = jnp.concatenate([pbtv, rows_c], axis=0)
        compact = _mm_bt(gh, datt)
        kc_col = compact[:, 4:5]
        kc_row = jnp.transpose(kc_col)
        rank_row, _ = _rank_desc(kc_col, kc_row)
        p2 = _hot(_isub(CAP) == rank_row)
        ordered = _mm(p2, compact)
        ordered_t = jnp.transpose(ordered)
        bx_scr[pl.ds(c, 1), :, :] = ordered[jnp.newaxis, :, :4]
        bt_scr[pl.ds(c, 1), :, :] = ordered_t[jnp.newaxis]
        return 0

    lax.fori_loop(0, CC, per_class, 0)
    bxv = bx_scr[...]
    btv = bt_scr[...]
    y1c, x1c = bxv[:, :, 0:1], bxv[:, :, 1:2]
    y2c, x2c = bxv[:, :, 2:3], bxv[:, :, 3:4]
    y1r, x1r = btv[:, 0:1, :], btv[:, 1:2, :]
    y2r, x2r = btv[:, 2:3, :], btv[:, 3:4, :]
    area_c = jnp.maximum(y2c - y1c, 0.0) * jnp.maximum(x2c - x1c, 0.0)
    area_r = jnp.maximum(y2r - y1r, 0.0) * jnp.maximum(x2r - x1r, 0.0)
    inter = (jnp.maximum(jnp.minimum(y2c, y2r) - jnp.maximum(y1c, y1r), 0.0)
             * jnp.maximum(jnp.minimum(x2c, x2r) - jnp.maximum(x1c, x1r), 0.0))
    iou_scr[...] = inter / jnp.maximum(area_c + area_r - inter, 1e-8)

    lane = lax.broadcasted_iota(jnp.int32, (CC, CAP), 1)

    def step(i, supp):
        hit = jnp.sum(jnp.where((lane == i) & (supp > 0), 1, 0),
                      axis=1, keepdims=True)
        kept = hit == 0
        iou_row = iou_scr[:, pl.ds(i, 1), :].reshape(CC, CAP)
        return supp | (kept & (iou_row > IOU_T) & (lane > i)).astype(jnp.int32)

    supp = lax.fori_loop(0, CAP, step, jnp.zeros((CC, CAP), jnp.int32))
    keep = supp == 0
    keep = keep & (_cumsum_lanes(keep.astype(jnp.int32)) <= TOPK)
    sel_scores = jnp.where(keep, btv[:, 4, :], jnp.float32(NEG))
    for c in range(CC):
        out_bt[0, 0:4, c * CAP:(c + 1) * CAP] = btv[c, 0:4, :]
        out_bt[0, 4:5, c * CAP:(c + 1) * CAP] = sel_scores[c:c + 1, :]


def _final_body(fdt, out):
    dt = fdt[0]
    row = dt[4:5, :]
    key = jnp.where(row == jnp.float32(NEG),
                    jnp.int32(-1),
                    lax.bitcast_convert_type(row, jnp.int32))
    sel, dest = _topk_select(key, TOPK, -1, _HI_KEY, 31)
    cls_row = (_ilane(NFLAT) // CAP).astype(jnp.float32)
    datt = jnp.concatenate([dt, cls_row], axis=0)
    gh = _hot((_isub(TOPK) == dest) & sel)
    compact = _mm_bt(gh, datt)
    kc_col = compact[:, 4:5]
    kc_row = jnp.transpose(kc_col)
    rank_row, _ = _rank_desc(kc_col, kc_row)
    p2 = _hot(_isub(TOPK) == rank_row)
    ordered = _mm(p2, compact)
    mask = _hot(ordered[:, 4:5] > BOX_SCORE)
    out[0] = ordered * mask


def kernel(cls_score_0, bbox_pred_0, cls_score_1, bbox_pred_1,
           cls_score_2, bbox_pred_2, origin_shapes):
    del origin_shapes
    c0 = cls_score_0.reshape(B, 4096, C)
    b0 = bbox_pred_0.reshape(B, 4096, 4 * (REG + 1))
    c1 = cls_score_1.reshape(B, 1024, C)
    b1 = bbox_pred_1.reshape(B, 1024, 4 * (REG + 1))
    c2 = cls_score_2.reshape(B, 256, C)
    b2 = bbox_pred_2.reshape(B, 256, 4 * (REG + 1))

    pool_s, pool_b = pl.pallas_call(
        _pool_body,
        grid=(B,),
        in_specs=[
            pl.BlockSpec((1, 4096, C), lambda i: (i, 0, 0)),
            pl.BlockSpec((1, 4096, 32), lambda i: (i, 0, 0)),
            pl.BlockSpec((1, 1024, C), lambda i: (i, 0, 0)),
            pl.BlockSpec((1, 1024, 32), lambda i: (i, 0, 0)),
            pl.BlockSpec((1, 256, C), lambda i: (i, 0, 0)),
            pl.BlockSpec((1, 256, 32), lambda i: (i, 0, 0)),
        ],
        out_specs=[
            pl.BlockSpec((1, NPOOL, C), lambda i: (i, 0, 0)),
            pl.BlockSpec((1, NPOOL, 4), lambda i: (i, 0, 0)),
        ],
        out_shape=[
            jax.ShapeDtypeStruct((B, NPOOL, C), jnp.float32),
            jax.ShapeDtypeStruct((B, NPOOL, 4), jnp.float32),
        ],
        scratch_shapes=[pltpu.VMEM((NMS_PRE, C + 5), jnp.float32)],
    )(c0, b0, c1, b1, c2, b2)

    pool_st = pool_s.transpose(0, 2, 1)
    pool_bt = pool_b.transpose(0, 2, 1)

    flat_bt = pl.pallas_call(
        _nms_body,
        grid=(B, C // CC),
        in_specs=[
            pl.BlockSpec((1, CC, NPOOL), lambda i, k: (i, k, 0)),
            pl.BlockSpec((1, 4, NPOOL), lambda i, k: (i, 0, 0)),
        ],
        out_specs=pl.BlockSpec((1, 5, CC * CAP), lambda i, k: (i, 0, k)),
        out_shape=jax.ShapeDtypeStruct((B, 5, NFLAT), jnp.float32),
        scratch_shapes=[
            pltpu.VMEM((CC, NPOOL), jnp.int32),
            pltpu.VMEM((CC, NPOOL), jnp.int32),
            pltpu.VMEM((CC, CAP, 4), jnp.float32),
            pltpu.VMEM((CC, 5, CAP), jnp.float32),
            pltpu.VMEM((CC, CAP, CAP), jnp.float32),
        ],
    )(pool_st, pool_bt)

    res = pl.pallas_call(
        _final_body,
        grid=(B,),
        in_specs=[pl.BlockSpec((1, 5, NFLAT), lambda i: (i, 0, 0))],
        out_specs=pl.BlockSpec((1, TOPK, 6), lambda i: (i, 0, 0)),
        out_shape=jax.ShapeDtypeStruct((B, TOPK, 6), jnp.float32),
    )(flat_bt)
    return res
